# Initial kernel scaffold; baseline (speedup 1.0000x reference)
import jax
import jax.numpy as jnp
from jax.experimental import pallas as pl


def kernel(events, mlp_W1, mlp_b1, mlp_W2, mlp_b2, mlp_W3, mlp_b3, rnn_l1_W, rnn_l1_b, lstm_Wih, lstm_Whh, lstm_bih, lstm_bhh, rnn_l2_W, rnn_l2_b):
    raise NotImplementedError("write your pallas kernel here")


# speedup vs baseline: 2.9794x; 2.9794x over previous
import functools

import jax
import jax.numpy as jnp
from jax import lax
from jax.experimental import pallas as pl
from jax.experimental.pallas import tpu as pltpu
from jax.experimental.pallas import tpu_sc as plsc

H = 180
W = 240
TAS = 3
NB = 4
MAXK = 8
N = 600000
HID = 20

B = NB * TAS
R = B * 2 * H * W
RPAD = R + 16
VOXN = MAXK * RPAD
DUMMY = R

ET = 2048
NET = 294
NPAD = ET * NET

NC = 2
NS = 16
NW = NC * NS
EVW = NPAD // NW
CH = 128
NCH = EVW // CH
SC_INNER = 7

T3 = 2560
G3 = R // T3


def _leaky(x):
    return jnp.where(x >= 0, x, 0.1 * x)


def _phase1_body(ev_ref, w1_ref, b1_ref, w2_ref, b2_ref, w3_ref, b3_ref,
                 val_ref, lin_ref):
    j = pl.program_id(0)
    ev = ev_ref[...]
    xc = ev[0:1, :].astype(jnp.int32)
    yc = ev[1:2, :].astype(jnp.int32)
    pc = ev[2:3, :].astype(jnp.int32)
    t = ev[3:4, :]
    ipc = ev[4:5, :].astype(jnp.int32)
    itc = ev[5:6, :].astype(jnp.int32)
    bc = ev[6:7, :].astype(jnp.int32)

    row = xc + W * yc + (W * H) * pc + (W * H * 2) * (bc * TAS + itc)
    lin = (ipc - 1) * RPAD + row
    col = j * ET + lax.broadcasted_iota(jnp.int32, (1, ET), 1)
    lin = jnp.where(col < N, lin, DUMMY)
    lin_ref[...] = lin

    h1 = _leaky(w1_ref[...] * t + b1_ref[...])
    h2 = _leaky(jnp.dot(w2_ref[...], h1,
                        preferred_element_type=jnp.float32) + b2_ref[...])
    val = jnp.dot(w3_ref[...], h2,
                  preferred_element_type=jnp.float32) + b3_ref[...]
    val_ref[...] = val


def _sc_scatter_body(lin_ref, val_ref, vox_ref, idx_v, val_v, sem):
    wid = lax.axis_index("c") * NS + lax.axis_index("s")
    pltpu.sync_copy(lin_ref.at[wid], idx_v)
    pltpu.sync_copy(val_ref.at[wid], val_v)

    @pl.loop(0, NCH, step=SC_INNER)
    def _(ch):
        copies = []
        for q in range(SC_INNER):
            copies.append(pltpu.async_copy(
                val_v.at[ch + q], vox_ref.at[idx_v.at[ch + q]], sem))
        for cp in copies:
            cp.wait()


def _phase3_body(vox_ref, whh_ref, u_ref, v_ref, w2_ref, b2_ref, out_ref):
    vb = vox_ref[...]
    length = jnp.sum((vb != 0.0).astype(jnp.int32), axis=0,
                     keepdims=True)
    rowsum = jnp.sum(vb, axis=0, keepdims=True)
    maxlen = jnp.max(length)

    whh = whh_ref[...]
    u = u_ref[...]
    v = v_ref[...]

    def step(k, carry):
        h, c = carry
        x = vox_ref[pl.ds(k, 1), :]
        gates = (jnp.dot(whh, h, preferred_element_type=jnp.float32)
                 + u * x + v)
        i = jax.nn.sigmoid(gates[0 * HID:1 * HID])
        f = jax.nn.sigmoid(gates[1 * HID:2 * HID])
        g = jnp.tanh(gates[2 * HID:3 * HID])
        o = jax.nn.sigmoid(gates[3 * HID:4 * HID])
        c_new = f * c + i * g
        h_new = o * jnp.tanh(c_new)
        m = k < length
        return jnp.where(m, h_new, h), jnp.where(m, c_new, c)

    h0 = jnp.zeros((HID, T3), jnp.float32)
    c0 = jnp.zeros((HID, T3), jnp.float32)
    hT, _ = lax.fori_loop(0, maxlen, step, (h0, c0))
    out = jnp.dot(w2_ref[...], hT, preferred_element_type=jnp.float32) \
        + b2_ref[...]
    out_ref[...] = jnp.where(rowsum != 0.0, out, 0.0)


def kernel(events, mlp_W1, mlp_b1, mlp_W2, mlp_b2, mlp_W3, mlp_b3,
           rnn_l1_W, rnn_l1_b, lstm_Wih, lstm_Whh, lstm_bih, lstm_bhh,
           rnn_l2_W, rnn_l2_b):
    f32 = jnp.float32
    ev = events.astype(f32)

    evT = jnp.zeros((7, NPAD), f32).at[:, :N].set(ev.T)
    w1c = mlp_W1.reshape(HID, 1)
    b1c = mlp_b1.reshape(HID, 1)
    b2c = mlp_b2.reshape(HID, 1)
    w3r = mlp_W3.reshape(1, HID)
    b3c = mlp_b3.reshape(1, 1)

    rep = lambda shape: pl.BlockSpec(shape, lambda j: (0, 0))
    val, lin = pl.pallas_call(
        _phase1_body,
        grid=(NET,),
        in_specs=[
            pl.BlockSpec((7, ET), lambda j: (0, j)),
            rep((HID, 1)), rep((HID, 1)), rep((HID, HID)), rep((HID, 1)),
            rep((1, HID)), rep((1, 1)),
        ],
        out_specs=[
            pl.BlockSpec((1, ET), lambda j: (0, j)),
            pl.BlockSpec((1, ET), lambda j: (0, j)),
        ],
        out_shape=[
            jax.ShapeDtypeStruct((1, NPAD), f32),
            jax.ShapeDtypeStruct((1, NPAD), jnp.int32),
        ],
    )(evT, w1c, b1c, mlp_W2, b2c, w3r, b3c)

    lin3 = lin.reshape(NW, NCH, CH)
    val3 = val.reshape(NW, NCH, CH)
    vox_ref = jax.new_ref(jnp.zeros((VOXN,), f32))
    sc_scatter = pl.kernel(
        _sc_scatter_body,
        out_type=(),
        mesh=plsc.VectorSubcoreMesh(core_axis_name="c", subcore_axis_name="s",
                                    num_cores=NC, num_subcores=NS),
        scratch_types=[
            pltpu.VMEM((NCH, CH), jnp.int32),
            pltpu.VMEM((NCH, CH), f32),
            pltpu.SemaphoreType.DMA,
        ],
    )
    sc_scatter(lin3, val3, vox_ref)
    vox2 = vox_ref[...].reshape(MAXK, RPAD)

    w1v = rnn_l1_W.reshape(HID)
    u = (lstm_Wih @ w1v).reshape(4 * HID, 1)
    v = (lstm_Wih @ rnn_l1_b + lstm_bih + lstm_bhh).reshape(4 * HID, 1)

    gird = pl.pallas_call(
        _phase3_body,
        grid=(G3,),
        in_specs=[
            pl.BlockSpec((MAXK, T3), lambda j: (0, j)),
            rep((4 * HID, HID)), rep((4 * HID, 1)), rep((4 * HID, 1)),
            rep((1, HID)), rep((1, 1)),
        ],
        out_specs=pl.BlockSpec((1, T3), lambda j: (0, j)),
        out_shape=jax.ShapeDtypeStruct((1, R), f32),
    )(vox2, lstm_Whh, u, v, rnn_l2_W, rnn_l2_b.reshape(1, 1))

    return gird.reshape(B, 2, H, W)


# speedup vs baseline: 2.9802x; 1.0003x over previous
import functools

import jax
import jax.numpy as jnp
from jax import lax
from jax.experimental import pallas as pl
from jax.experimental.pallas import tpu as pltpu
from jax.experimental.pallas import tpu_sc as plsc

H = 180
W = 240
TAS = 3
NB = 4
MAXK = 8
N = 600000
HID = 20

B = NB * TAS
R = B * 2 * H * W
RPAD = R + 16
VOXN = MAXK * RPAD
DUMMY = R

ET = 2048
NET = 294
NPAD = ET * NET

NC = 2
NS = 16
NW = NC * NS
EVW = NPAD // NW
CH = 128
NCH = EVW // CH
SC_INNER = 7

T3 = 2560
G3 = R // T3


def _leaky(x):
    return jnp.where(x >= 0, x, 0.1 * x)


def _phase1_body(ev_ref, w1_ref, b1_ref, w2_ref, b2_ref, w3_ref, b3_ref,
                 val_ref, lin_ref):
    j = pl.program_id(0)
    ev = ev_ref[...]
    xc = ev[0:1, :].astype(jnp.int32)
    yc = ev[1:2, :].astype(jnp.int32)
    pc = ev[2:3, :].astype(jnp.int32)
    t = ev[3:4, :]
    ipc = ev[4:5, :].astype(jnp.int32)
    itc = ev[5:6, :].astype(jnp.int32)
    bc = ev[6:7, :].astype(jnp.int32)

    row = xc + W * yc + (W * H) * pc + (W * H * 2) * (bc * TAS + itc)
    lin = (ipc - 1) * RPAD + row
    col = j * ET + lax.broadcasted_iota(jnp.int32, (1, ET), 1)
    lin = jnp.where(col < N, lin, DUMMY)
    lin_ref[...] = lin

    h1 = _leaky(w1_ref[...] * t + b1_ref[...])
    h2 = _leaky(jnp.dot(w2_ref[...], h1,
                        preferred_element_type=jnp.float32) + b2_ref[...])
    val = jnp.dot(w3_ref[...], h2,
                  preferred_element_type=jnp.float32) + b3_ref[...]
    val_ref[...] = val


def _sc_scatter_body(lin_ref, val_ref, vox_ref, idx_v, val_v, sem):
    wid = lax.axis_index("c") * NS + lax.axis_index("s")
    pltpu.sync_copy(lin_ref.at[wid], idx_v)
    pltpu.sync_copy(val_ref.at[wid], val_v)

    @pl.loop(0, NCH)
    def _(ch):
        pltpu.async_copy(val_v.at[ch], vox_ref.at[idx_v.at[ch]], sem)

    pltpu.make_async_copy(val_ref.at[wid], val_v, sem).wait()


def _phase3_body(vox_ref, whh_ref, u_ref, v_ref, w2_ref, b2_ref, out_ref):
    vb = vox_ref[...]
    length = jnp.sum((vb != 0.0).astype(jnp.int32), axis=0,
                     keepdims=True)
    rowsum = jnp.sum(vb, axis=0, keepdims=True)
    maxlen = jnp.max(length)

    whh = whh_ref[...]
    u = u_ref[...]
    v = v_ref[...]

    def step(k, carry):
        h, c = carry
        x = vox_ref[pl.ds(k, 1), :]
        gates = (jnp.dot(whh, h, preferred_element_type=jnp.float32)
                 + u * x + v)
        i = jax.nn.sigmoid(gates[0 * HID:1 * HID])
        f = jax.nn.sigmoid(gates[1 * HID:2 * HID])
        g = jnp.tanh(gates[2 * HID:3 * HID])
        o = jax.nn.sigmoid(gates[3 * HID:4 * HID])
        c_new = f * c + i * g
        h_new = o * jnp.tanh(c_new)
        m = k < length
        return jnp.where(m, h_new, h), jnp.where(m, c_new, c)

    h0 = jnp.zeros((HID, T3), jnp.float32)
    c0 = jnp.zeros((HID, T3), jnp.float32)
    hT, _ = lax.fori_loop(0, maxlen, step, (h0, c0))
    out = jnp.dot(w2_ref[...], hT, preferred_element_type=jnp.float32) \
        + b2_ref[...]
    out_ref[...] = jnp.where(rowsum != 0.0, out, 0.0)


def kernel(events, mlp_W1, mlp_b1, mlp_W2, mlp_b2, mlp_W3, mlp_b3,
           rnn_l1_W, rnn_l1_b, lstm_Wih, lstm_Whh, lstm_bih, lstm_bhh,
           rnn_l2_W, rnn_l2_b):
    f32 = jnp.float32
    ev = events.astype(f32)

    evT = jnp.zeros((7, NPAD), f32).at[:, :N].set(ev.T)
    w1c = mlp_W1.reshape(HID, 1)
    b1c = mlp_b1.reshape(HID, 1)
    b2c = mlp_b2.reshape(HID, 1)
    w3r = mlp_W3.reshape(1, HID)
    b3c = mlp_b3.reshape(1, 1)

    rep = lambda shape: pl.BlockSpec(shape, lambda j: (0, 0))
    val, lin = pl.pallas_call(
        _phase1_body,
        grid=(NET,),
        in_specs=[
            pl.BlockSpec((7, ET), lambda j: (0, j)),
            rep((HID, 1)), rep((HID, 1)), rep((HID, HID)), rep((HID, 1)),
            rep((1, HID)), rep((1, 1)),
        ],
        out_specs=[
            pl.BlockSpec((1, ET), lambda j: (0, j)),
            pl.BlockSpec((1, ET), lambda j: (0, j)),
        ],
        out_shape=[
            jax.ShapeDtypeStruct((1, NPAD), f32),
            jax.ShapeDtypeStruct((1, NPAD), jnp.int32),
        ],
    )(evT, w1c, b1c, mlp_W2, b2c, w3r, b3c)

    lin3 = lin.reshape(NW, NCH, CH)
    val3 = val.reshape(NW, NCH, CH)
    vox_ref = jax.new_ref(jnp.zeros((VOXN,), f32))
    sc_scatter = pl.kernel(
        _sc_scatter_body,
        out_type=(),
        mesh=plsc.VectorSubcoreMesh(core_axis_name="c", subcore_axis_name="s",
                                    num_cores=NC, num_subcores=NS),
        scratch_types=[
            pltpu.VMEM((NCH, CH), jnp.int32),
            pltpu.VMEM((NCH, CH), f32),
            pltpu.SemaphoreType.DMA,
        ],
    )
    sc_scatter(lin3, val3, vox_ref)
    vox2 = vox_ref[...].reshape(MAXK, RPAD)

    w1v = rnn_l1_W.reshape(HID)
    u = (lstm_Wih @ w1v).reshape(4 * HID, 1)
    v = (lstm_Wih @ rnn_l1_b + lstm_bih + lstm_bhh).reshape(4 * HID, 1)

    gird = pl.pallas_call(
        _phase3_body,
        grid=(G3,),
        in_specs=[
            pl.BlockSpec((MAXK, T3), lambda j: (0, j)),
            rep((4 * HID, HID)), rep((4 * HID, 1)), rep((4 * HID, 1)),
            rep((1, HID)), rep((1, 1)),
        ],
        out_specs=pl.BlockSpec((1, T3), lambda j: (0, j)),
        out_shape=jax.ShapeDtypeStruct((1, R), f32),
    )(vox2, lstm_Whh, u, v, rnn_l2_W, rnn_l2_b.reshape(1, 1))

    return gird.reshape(B, 2, H, W)
